# baseline (speedup 1.0000x reference)
import functools

import jax
import jax.numpy as jnp
from jax import lax
from jax.experimental import pallas as pl
from jax.experimental.pallas import tpu as pltpu
from jax.experimental.pallas import tpu_sc as plsc

_NC = 2
_NS = 16
_NW = _NC * _NS


def _sc_gather(user_idx, item_idx, user_table, item_table, B, D):
    b_per_w = B // _NW
    mesh = plsc.VectorSubcoreMesh(core_axis_name="c", subcore_axis_name="s")

    @functools.partial(
        pl.kernel,
        mesh=mesh,
        out_type=[
            jax.ShapeDtypeStruct((B, D), jnp.float32),
            jax.ShapeDtypeStruct((B, D), jnp.float32),
        ],
        scratch_types=[
            pltpu.SMEM((b_per_w,), jnp.int32),
            pltpu.SMEM((b_per_w,), jnp.int32),
            pltpu.VMEM_SHARED((_NS, b_per_w), jnp.int32),
            pltpu.VMEM_SHARED((_NS, b_per_w), jnp.int32),
            pltpu.SemaphoreType.DMA,
        ],
    )
    def k(uidx_hbm, iidx_hbm, utab_hbm, itab_hbm, uout_hbm, iout_hbm,
          usmem, ismem, uidx_sp, iidx_sp, sem):
        sid = lax.axis_index("s")
        wid = sid * _NC + lax.axis_index("c")
        base = wid * b_per_w
        pltpu.sync_copy(uidx_hbm.at[pl.ds(base, b_per_w)], uidx_sp.at[sid])
        pltpu.sync_copy(iidx_hbm.at[pl.ds(base, b_per_w)], iidx_sp.at[sid])
        pltpu.sync_copy(uidx_sp.at[sid], usmem)
        pltpu.sync_copy(iidx_sp.at[sid], ismem)

        def fire(kk, carry):
            ur = usmem[kk]
            ir = ismem[kk]
            row = base + kk
            pltpu.make_async_copy(
                utab_hbm.at[pl.ds(ur, 1), :],
                uout_hbm.at[pl.ds(row, 1), :], sem).start()
            pltpu.make_async_copy(
                itab_hbm.at[pl.ds(ir, 1), :],
                iout_hbm.at[pl.ds(row, 1), :], sem).start()
            return carry

        lax.fori_loop(0, b_per_w, fire, 0)
        pltpu.make_async_copy(
            uout_hbm.at[pl.ds(0, b_per_w)],
            uout_hbm.at[pl.ds(base, b_per_w)], sem).wait()
        pltpu.make_async_copy(
            iout_hbm.at[pl.ds(0, b_per_w)],
            iout_hbm.at[pl.ds(base, b_per_w)], sem).wait()

    return k(user_idx, item_idx, user_table, item_table)


def _mlp_body(u_ref, i_ref, w1_ref, b1_ref, w2_ref, b2_ref,
              pw1_ref, pb1_ref, pw2_ref, pb2_ref, o_ref):
    x = jnp.concatenate([u_ref[...], i_ref[...]], axis=1)
    h = jnp.dot(x, w1_ref[...], preferred_element_type=jnp.float32)
    h = jnp.maximum(h + b1_ref[...], 0.0)
    logits = jnp.sum(h * w2_ref[...], axis=1, keepdims=True) + b2_ref[0, 0]
    a = jax.nn.sigmoid(logits)
    xw = x * a
    p = jnp.dot(xw, pw1_ref[...], preferred_element_type=jnp.float32)
    p = jnp.maximum(p + pb1_ref[...], 0.0)
    o_ref[...] = jnp.sum(p * pw2_ref[...], axis=1) + pb2_ref[0, 0]


def _tc_mlp(u, it, att_w1, att_b1, att_w2, att_b2,
            pred_w1, pred_b1, pred_w2, pred_b2):
    B, D = u.shape
    BLK = 2048
    full = lambda s: pl.BlockSpec(s, lambda i: (0,) * len(s))
    return pl.pallas_call(
        _mlp_body,
        grid=(B // BLK,),
        in_specs=[
            pl.BlockSpec((BLK, D), lambda i: (i, 0)),
            pl.BlockSpec((BLK, D), lambda i: (i, 0)),
            full(att_w1.shape),
            full(att_b1.shape),
            full(att_w2.shape),
            full(att_b2.shape),
            full(pred_w1.shape),
            full(pred_b1.shape),
            full(pred_w2.shape),
            full(pred_b2.shape),
        ],
        out_specs=pl.BlockSpec((BLK,), lambda i: (i,)),
        out_shape=jax.ShapeDtypeStruct((B,), jnp.float32),
    )(u, it, att_w1, att_b1, att_w2, att_b2,
      pred_w1, pred_b1, pred_w2, pred_b2)


def kernel(user_indices, item_indices, user_table, item_table,
           att_w1, att_b1, att_w2, att_b2,
           pred_w1, pred_b1, pred_w2, pred_b2):
    B = user_indices.shape[0]
    D = user_table.shape[1]
    uidx = user_indices.astype(jnp.int32)
    iidx = item_indices.astype(jnp.int32)
    u, it = _sc_gather(uidx, iidx, user_table, item_table, B, D)
    return _tc_mlp(
        u, it,
        att_w1, att_b1.reshape(1, -1),
        att_w2.reshape(1, -1), att_b2.reshape(1, 1),
        pred_w1, pred_b1.reshape(1, -1),
        pred_w2.reshape(1, -1), pred_b2.reshape(1, 1),
    )


# speedup vs baseline: 1.2581x; 1.2581x over previous
import functools

import jax
import jax.numpy as jnp
from jax import lax
from jax.experimental import pallas as pl
from jax.experimental.pallas import tpu as pltpu
from jax.experimental.pallas import tpu_sc as plsc

_NC = 2
_NS = 16
_NW = _NC * _NS


def _sc_gather(user_idx, item_idx, user_table, item_table, B, D):
    b_per_w = B // _NW
    mesh = plsc.VectorSubcoreMesh(core_axis_name="c", subcore_axis_name="s")

    @functools.partial(
        pl.kernel,
        mesh=mesh,
        out_type=[
            jax.ShapeDtypeStruct((B, D), jnp.float32),
            jax.ShapeDtypeStruct((B, D), jnp.float32),
        ],
        scratch_types=[
            pltpu.SMEM((b_per_w,), jnp.int32),
            pltpu.SMEM((b_per_w,), jnp.int32),
            pltpu.VMEM_SHARED((_NS, b_per_w), jnp.int32),
            pltpu.VMEM_SHARED((_NS, b_per_w), jnp.int32),
            pltpu.SemaphoreType.DMA,
        ],
    )
    def k(uidx_hbm, iidx_hbm, utab_hbm, itab_hbm, uout_hbm, iout_hbm,
          usmem, ismem, uidx_sp, iidx_sp, sem):
        sid = lax.axis_index("s")
        wid = sid * _NC + lax.axis_index("c")
        base = wid * b_per_w
        pltpu.sync_copy(uidx_hbm.at[pl.ds(base, b_per_w)], uidx_sp.at[sid])
        pltpu.sync_copy(iidx_hbm.at[pl.ds(base, b_per_w)], iidx_sp.at[sid])
        pltpu.sync_copy(uidx_sp.at[sid], usmem)
        pltpu.sync_copy(iidx_sp.at[sid], ismem)

        def fire(kk, carry):
            ur = usmem[kk]
            ir = ismem[kk]
            row = base + kk
            pltpu.make_async_copy(
                utab_hbm.at[ur >> 3, pl.ds(ur & 7, 1), :],
                uout_hbm.at[pl.ds(row, 1), :], sem).start()
            pltpu.make_async_copy(
                itab_hbm.at[ir >> 3, pl.ds(ir & 7, 1), :],
                iout_hbm.at[pl.ds(row, 1), :], sem).start()
            return carry

        lax.fori_loop(0, b_per_w, fire, 0)
        pltpu.make_async_copy(
            uout_hbm.at[pl.ds(0, b_per_w)],
            uout_hbm.at[pl.ds(base, b_per_w)], sem).wait()
        pltpu.make_async_copy(
            iout_hbm.at[pl.ds(0, b_per_w)],
            iout_hbm.at[pl.ds(base, b_per_w)], sem).wait()

    return k(user_idx, item_idx, user_table, item_table)


def _mlp_body(u_ref, i_ref, w1_ref, b1_ref, w2_ref, b2_ref,
              pw1_ref, pb1_ref, pw2_ref, pb2_ref, o_ref):
    x = jnp.concatenate([u_ref[...], i_ref[...]], axis=1)
    h = jnp.dot(x, w1_ref[...], preferred_element_type=jnp.float32)
    h = jnp.maximum(h + b1_ref[...], 0.0)
    logits = jnp.sum(h * w2_ref[...], axis=1, keepdims=True) + b2_ref[0, 0]
    a = jax.nn.sigmoid(logits)
    xw = x * a
    p = jnp.dot(xw, pw1_ref[...], preferred_element_type=jnp.float32)
    p = jnp.maximum(p + pb1_ref[...], 0.0)
    o_ref[...] = jnp.sum(p * pw2_ref[...], axis=1) + pb2_ref[0, 0]


def _tc_mlp(u, it, att_w1, att_b1, att_w2, att_b2,
            pred_w1, pred_b1, pred_w2, pred_b2):
    B, D = u.shape
    BLK = 2048
    full = lambda s: pl.BlockSpec(s, lambda i: (0,) * len(s))
    return pl.pallas_call(
        _mlp_body,
        grid=(B // BLK,),
        in_specs=[
            pl.BlockSpec((BLK, D), lambda i: (i, 0)),
            pl.BlockSpec((BLK, D), lambda i: (i, 0)),
            full(att_w1.shape),
            full(att_b1.shape),
            full(att_w2.shape),
            full(att_b2.shape),
            full(pred_w1.shape),
            full(pred_b1.shape),
            full(pred_w2.shape),
            full(pred_b2.shape),
        ],
        out_specs=pl.BlockSpec((BLK,), lambda i: (i,)),
        out_shape=jax.ShapeDtypeStruct((B,), jnp.float32),
    )(u, it, att_w1, att_b1, att_w2, att_b2,
      pred_w1, pred_b1, pred_w2, pred_b2)


def kernel(user_indices, item_indices, user_table, item_table,
           att_w1, att_b1, att_w2, att_b2,
           pred_w1, pred_b1, pred_w2, pred_b2):
    B = user_indices.shape[0]
    D = user_table.shape[1]
    uidx = user_indices.astype(jnp.int32)
    iidx = item_indices.astype(jnp.int32)
    N = user_table.shape[0]
    u, it = _sc_gather(
        uidx, iidx,
        user_table.reshape(N // 8, 8, D),
        item_table.reshape(N // 8, 8, D),
        B, D)
    return _tc_mlp(
        u, it,
        att_w1, att_b1.reshape(1, -1),
        att_w2.reshape(1, -1), att_b2.reshape(1, 1),
        pred_w1, pred_b1.reshape(1, -1),
        pred_w2.reshape(1, -1), pred_b2.reshape(1, 1),
    )
